# Initial kernel scaffold; baseline (speedup 1.0000x reference)
import jax
import jax.numpy as jnp
from jax.experimental import pallas as pl


def kernel(x, edge_index, edge_weight, W, b):
    raise NotImplementedError("write your pallas kernel here")


# speedup vs baseline: 3.8496x; 3.8496x over previous
import jax
import jax.numpy as jnp
from jax import lax
from jax.experimental import pallas as pl
from jax.experimental.pallas import tpu as pltpu
from jax.experimental.pallas import tpu_sc as plsc

N_NODES = 10000
N_EDGES = 320000
IN_CH = 128
OUT_CH = 128

NC = 2
NS = 16
L = 16
NW = NC * NS

CHUNK = 128
CHUNKS_PER_TILE = 79
EDGES_PER_TILE = CHUNKS_PER_TILE * CHUNK
E_PAD = NW * EDGES_PER_TILE
N_PAD = 10240
ROWS_PER_TILE = N_PAD // NS
SLAB = 128

MM_BLOCK = 1000


def _mm_body(x_ref, w_ref, h_ref):
    h_ref[...] = lax.dot_general(x_ref[...], w_ref[...],
                                 (((1,), (1,)), ((), ())),
                                 preferred_element_type=jnp.float32)


def _matmul(x, W):
    return pl.pallas_call(
        _mm_body,
        grid=(N_NODES // MM_BLOCK,),
        in_specs=[
            pl.BlockSpec((MM_BLOCK, IN_CH), lambda i: (i, 0)),
            pl.BlockSpec((OUT_CH, IN_CH), lambda i: (0, 0)),
        ],
        out_specs=pl.BlockSpec((MM_BLOCK, OUT_CH), lambda i: (i, 0)),
        out_shape=jax.ShapeDtypeStruct((N_NODES, OUT_CH), jnp.float32),
    )(x, W)


def _comb_body(p0_ref, p1_ref, b_ref, o_ref):
    o_ref[...] = p0_ref[...] + p1_ref[...] + b_ref[...]


def _combine(p0, p1, b):
    return pl.pallas_call(
        _comb_body,
        grid=(N_NODES // MM_BLOCK,),
        in_specs=[
            pl.BlockSpec((MM_BLOCK, OUT_CH), lambda i: (i, 0)),
            pl.BlockSpec((MM_BLOCK, OUT_CH), lambda i: (i, 0)),
            pl.BlockSpec((1, OUT_CH), lambda i: (0, 0)),
        ],
        out_specs=pl.BlockSpec((MM_BLOCK, OUT_CH), lambda i: (i, 0)),
        out_shape=jax.ShapeDtypeStruct((N_NODES, OUT_CH), jnp.float32),
    )(p0, p1, b)


def _sc_body(h_hbm, src_hbm, dst_hbm, w_hbm, out_hbm,
             idx_sb, idx_db, wvb, rows, acc, sem):
    cid = lax.axis_index("c")
    sid = lax.axis_index("s")
    wid = cid * NS + sid

    zeros16 = jnp.zeros((L,), jnp.float32)

    def _zero_row(r, _):
        for q in range(OUT_CH // L):
            rows[r, pl.ds(q * L, L)] = zeros16
        return 0

    lax.fori_loop(0, SLAB, _zero_row, 0)
    for k in range(ROWS_PER_TILE // SLAB):
        pltpu.sync_copy(rows, acc.at[pl.ds(sid * ROWS_PER_TILE + k * SLAB, SLAB)])
    plsc.subcore_barrier()

    dnums = lax.GatherDimensionNumbers(
        offset_dims=(), collapsed_slice_dims=(0,), start_index_map=(0,))

    def _splat(w16, u):
        return lax.gather(w16, jnp.full((L, 1), u, jnp.int32), dnums, (1,),
                          mode=lax.GatherScatterMode.PROMISE_IN_BOUNDS)

    def _chunk(j, _):
        pltpu.sync_copy(src_hbm.at[wid, j], idx_sb)
        pltpu.sync_copy(dst_hbm.at[wid, j], idx_db)
        pltpu.sync_copy(w_hbm.at[wid, j], wvb)
        pltpu.async_copy(h_hbm.at[idx_sb], rows, sem).wait()

        def _group(g, _):
            w16 = wvb[pl.ds(g * L, L)]
            for u in range(L):
                e = g * L + u
                ws = _splat(w16, u)
                for q in range(OUT_CH // L):
                    rows[e, pl.ds(q * L, L)] = rows[e, pl.ds(q * L, L)] * ws
            return 0

        lax.fori_loop(0, CHUNK // L, _group, 0)
        pltpu.sync_copy(rows, acc.at[idx_db], add=True)
        return 0

    lax.fori_loop(0, CHUNKS_PER_TILE, _chunk, 0)
    plsc.subcore_barrier()

    for k in range(ROWS_PER_TILE // SLAB):
        base = sid * ROWS_PER_TILE + k * SLAB
        pltpu.sync_copy(acc.at[pl.ds(base, SLAB)], rows)
        pltpu.sync_copy(rows, out_hbm.at[cid, pl.ds(base, SLAB)])


_scatter_gather = pl.kernel(
    _sc_body,
    out_type=jax.ShapeDtypeStruct((NC, N_PAD, OUT_CH), jnp.float32),
    mesh=plsc.VectorSubcoreMesh(core_axis_name="c", subcore_axis_name="s",
                                num_cores=NC, num_subcores=NS),
    scratch_types=[
        pltpu.VMEM((CHUNK,), jnp.int32),
        pltpu.VMEM((CHUNK,), jnp.int32),
        pltpu.VMEM((CHUNK,), jnp.float32),
        pltpu.VMEM((CHUNK, OUT_CH), jnp.float32),
        pltpu.VMEM_SHARED((N_PAD, OUT_CH), jnp.float32),
        pltpu.SemaphoreType.DMA,
    ],
)


def kernel(x, edge_index, edge_weight, W, b):
    h = _matmul(x, W)
    pad = E_PAD - N_EDGES
    src = jnp.concatenate(
        [edge_index[0].astype(jnp.int32), jnp.zeros((pad,), jnp.int32)]
    ).reshape(NW, CHUNKS_PER_TILE, CHUNK)
    dst = jnp.concatenate(
        [edge_index[1].astype(jnp.int32), jnp.zeros((pad,), jnp.int32)]
    ).reshape(NW, CHUNKS_PER_TILE, CHUNK)
    wgt = jnp.concatenate(
        [edge_weight.astype(jnp.float32), jnp.zeros((pad,), jnp.float32)]
    ).reshape(NW, CHUNKS_PER_TILE, CHUNK)
    parts = _scatter_gather(h, src, dst, wgt)
    return _combine(parts[0, :N_NODES], parts[1, :N_NODES], b.reshape(1, OUT_CH))
